# Initial kernel scaffold; baseline (speedup 1.0000x reference)
import jax
import jax.numpy as jnp
from jax.experimental import pallas as pl


def kernel(x, edge_index, W1, W2, W3, b1, b2, b3, lin_W, lin_b):
    raise NotImplementedError("write your pallas kernel here")


# speedup vs baseline: 72.2839x; 72.2839x over previous
import jax
import jax.numpy as jnp
from jax.experimental import pallas as pl
from jax.experimental.pallas import tpu as pltpu

_N = 4096
_H = 64
_NCLS = 10
_IB = 512
_NI = _N // _IB


def _gcn_fused_kernel(A_ref, x_ref, W1_ref, W2_ref, W3_ref,
                      b1_ref, b2_ref, b3_ref,
                      LW1_ref, LW2_ref, LW3_ref, lb_ref,
                      out_ref, B_scr, h1_scr, h2_scr):
    p = pl.program_id(0)
    i = pl.program_id(1)

    @pl.when(jnp.logical_and(p == 0, i == 0))
    def _():
        B_scr[...] = jnp.dot(x_ref[...], W1_ref[...],
                             preferred_element_type=jnp.float32)

    @pl.when(jnp.logical_and(p == 1, i == 0))
    def _():
        B_scr[...] = jnp.dot(h1_scr[...], W2_ref[...],
                             preferred_element_type=jnp.float32)

    @pl.when(jnp.logical_and(p == 2, i == 0))
    def _():
        B_scr[...] = jnp.dot(h2_scr[...], W3_ref[...],
                             preferred_element_type=jnp.float32)

    agg = jax.lax.dot_general(A_ref[...], B_scr[...],
                              (((0,), (0,)), ((), ())),
                              preferred_element_type=jnp.float32)

    @pl.when(p == 0)
    def _():
        h1_scr[pl.ds(i * _IB, _IB), :] = jnp.maximum(agg + b1_ref[...], 0.0)

    @pl.when(p == 1)
    def _():
        h2_scr[pl.ds(i * _IB, _IB), :] = jnp.maximum(agg + b2_ref[...], 0.0)

    @pl.when(p == 2)
    def _():
        x3 = agg + b3_ref[...]
        x1 = h1_scr[pl.ds(i * _IB, _IB), :]
        x2 = h2_scr[pl.ds(i * _IB, _IB), :]
        logits = (jnp.dot(x1, LW1_ref[...], preferred_element_type=jnp.float32)
                  + jnp.dot(x2, LW2_ref[...], preferred_element_type=jnp.float32)
                  + jnp.dot(x3, LW3_ref[...], preferred_element_type=jnp.float32)
                  + lb_ref[...])
        m = jnp.max(logits, axis=1, keepdims=True)
        s = logits - m
        lse = jnp.log(jnp.sum(jnp.exp(s), axis=1, keepdims=True))
        out_ref[...] = s - lse


def kernel(x, edge_index, W1, W2, W3, b1, b2, b3, lin_W, lin_b):
    n, d_in = x.shape
    A = edge_index
    full = lambda r, c: pl.BlockSpec((r, c), lambda p, i: (0, 0))
    out = pl.pallas_call(
        _gcn_fused_kernel,
        grid=(3, _NI),
        in_specs=[
            pl.BlockSpec((_N, _IB), lambda p, i: (0, i)),
            full(_N, d_in),
            full(d_in, _H), full(_H, _H), full(_H, _H),
            full(1, _H), full(1, _H), full(1, _H),
            full(_H, _NCLS), full(_H, _NCLS), full(_H, _NCLS),
            full(1, _NCLS),
        ],
        out_specs=pl.BlockSpec((_IB, _NCLS), lambda p, i: (i, 0)),
        out_shape=jax.ShapeDtypeStruct((_N, _NCLS), jnp.float32),
        scratch_shapes=[
            pltpu.VMEM((_N, _H), jnp.float32),
            pltpu.VMEM((_N, _H), jnp.float32),
            pltpu.VMEM((_N, _H), jnp.float32),
        ],
    )(
        A, x, W1, W2, W3,
        b1.reshape(1, _H), b2.reshape(1, _H), b3.reshape(1, _H),
        lin_W[:_H], lin_W[_H:2 * _H], lin_W[2 * _H:],
        lin_b.reshape(1, _NCLS),
    )
    return out


# speedup vs baseline: 77.2883x; 1.0692x over previous
import jax
import jax.numpy as jnp
from jax.experimental import pallas as pl
from jax.experimental.pallas import tpu as pltpu

_N = 4096
_H = 64
_NCLS = 10
_IB = 512
_NI = _N // _IB


def _gcn_fused_kernel(A_ref, x_ref, W1_ref, W2_ref, W3_ref,
                      b1_ref, b2_ref, b3_ref,
                      LW1_ref, LW2_ref, LW3_ref, lb_ref,
                      out_ref, B_scr, h1_scr, h2_scr):
    p = pl.program_id(0)
    i = pl.program_id(1)

    @pl.when(jnp.logical_and(p == 0, i == 0))
    def _():
        B_scr[...] = jnp.dot(x_ref[...], W1_ref[...],
                             preferred_element_type=jnp.float32)

    @pl.when(jnp.logical_and(p == 1, i == 0))
    def _():
        B_scr[...] = jnp.dot(h1_scr[...], W2_ref[...],
                             preferred_element_type=jnp.float32)

    @pl.when(jnp.logical_and(p == 2, i == 0))
    def _():
        B_scr[...] = jnp.dot(h2_scr[...], W3_ref[...],
                             preferred_element_type=jnp.float32)

    agg = jax.lax.dot_general(A_ref[...].astype(jnp.bfloat16),
                              B_scr[...].astype(jnp.bfloat16),
                              (((0,), (0,)), ((), ())),
                              preferred_element_type=jnp.float32)

    @pl.when(p == 0)
    def _():
        h1_scr[pl.ds(i * _IB, _IB), :] = jnp.maximum(agg + b1_ref[...], 0.0)

    @pl.when(p == 1)
    def _():
        h2_scr[pl.ds(i * _IB, _IB), :] = jnp.maximum(agg + b2_ref[...], 0.0)

    @pl.when(p == 2)
    def _():
        x3 = agg + b3_ref[...]
        x1 = h1_scr[pl.ds(i * _IB, _IB), :]
        x2 = h2_scr[pl.ds(i * _IB, _IB), :]
        logits = (jnp.dot(x1, LW1_ref[...], preferred_element_type=jnp.float32)
                  + jnp.dot(x2, LW2_ref[...], preferred_element_type=jnp.float32)
                  + jnp.dot(x3, LW3_ref[...], preferred_element_type=jnp.float32)
                  + lb_ref[...])
        m = jnp.max(logits, axis=1, keepdims=True)
        s = logits - m
        lse = jnp.log(jnp.sum(jnp.exp(s), axis=1, keepdims=True))
        out_ref[...] = s - lse


def kernel(x, edge_index, W1, W2, W3, b1, b2, b3, lin_W, lin_b):
    n, d_in = x.shape
    A = edge_index
    full = lambda r, c: pl.BlockSpec((r, c), lambda p, i: (0, 0))
    out = pl.pallas_call(
        _gcn_fused_kernel,
        grid=(3, _NI),
        in_specs=[
            pl.BlockSpec((_N, _IB), lambda p, i: (0, i)),
            full(_N, d_in),
            full(d_in, _H), full(_H, _H), full(_H, _H),
            full(1, _H), full(1, _H), full(1, _H),
            full(_H, _NCLS), full(_H, _NCLS), full(_H, _NCLS),
            full(1, _NCLS),
        ],
        out_specs=pl.BlockSpec((_IB, _NCLS), lambda p, i: (i, 0)),
        out_shape=jax.ShapeDtypeStruct((_N, _NCLS), jnp.float32),
        scratch_shapes=[
            pltpu.VMEM((_N, _H), jnp.float32),
            pltpu.VMEM((_N, _H), jnp.float32),
            pltpu.VMEM((_N, _H), jnp.float32),
        ],
    )(
        A, x, W1, W2, W3,
        b1.reshape(1, _H), b2.reshape(1, _H), b3.reshape(1, _H),
        lin_W[:_H], lin_W[_H:2 * _H], lin_W[2 * _H:],
        lin_b.reshape(1, _NCLS),
    )
    return out
